# baseline (speedup 1.0000x reference)
import functools

import jax
import jax.numpy as jnp
from jax import lax
from jax.experimental import pallas as pl
from jax.experimental.pallas import tpu as pltpu
from jax.experimental.pallas import tpu_sc as plsc

_N = 10000
_E = 320000
_D = 128

_NC = 2
_NS = 16
_NW = _NC * _NS
_EPW = _E // _NW
_CH = 80
_NCHUNK = _EPW // _CH
_NSUP = 5
_CPS = _NCHUNK // _NSUP

_STRIPE_BOUNDS = [8 * ((_N // 8) * i // _NS) for i in range(_NS)] + [_N]
_STRIPES = [(_STRIPE_BOUNDS[i], _STRIPE_BOUNDS[i + 1] - _STRIPE_BOUNDS[i])
            for i in range(_NS)]

_mesh = plsc.VectorSubcoreMesh(core_axis_name="c", subcore_axis_name="s")


@functools.partial(
    pl.kernel,
    out_type=jax.ShapeDtypeStruct((_NC, _N), jnp.float32),
    mesh=_mesh,
    scratch_types=[
        pltpu.VMEM((_NSUP, _CPS, _CH), jnp.int32),
        pltpu.VMEM((_CH,), jnp.float32),
        pltpu.VMEM((_N,), jnp.float32),
        pltpu.VMEM_SHARED((_N,), jnp.float32),
        pltpu.SemaphoreType.DMA,
    ],
)
def _deg_kernel(dst_hbm, deg_hbm, dst_v, ones_v, zeros_v, deg_acc, sem_d):
    c = lax.axis_index("c")
    s = lax.axis_index("s")
    wid = c * _NS + s
    pltpu.sync_copy(dst_hbm.at[wid], dst_v)
    for k in range(_CH // 16):
        ones_v[pl.ds(k * 16, 16)] = jnp.ones((16,), jnp.float32)

    @pl.when(s == 0)
    def _zero():
        def zbody(i, carry):
            zeros_v[pl.ds(i * 16, 16)] = jnp.zeros((16,), jnp.float32)
            return carry

        lax.fori_loop(0, _N // 16, zbody, 0)
        pltpu.sync_copy(zeros_v, deg_acc)

    plsc.subcore_barrier()

    for sup in range(_NSUP):
        def body(j, carry, sup=sup):
            pltpu.make_async_copy(
                ones_v, deg_acc.at[dst_v.at[sup, j]], sem_d).start(add=True)
            return carry

        lax.fori_loop(0, _CPS, body, 0)

    def drain(j, carry):
        pltpu.make_async_copy(
            ones_v, deg_acc.at[dst_v.at[0, 0]], sem_d).wait()
        return carry

    lax.fori_loop(0, _NCHUNK, drain, 0)
    plsc.subcore_barrier()

    @pl.when(s == 0)
    def _out():
        pltpu.sync_copy(deg_acc, deg_hbm.at[c])


@functools.partial(
    pl.kernel,
    out_type=jax.ShapeDtypeStruct((_NC, _N, _D), jnp.float32),
    mesh=_mesh,
    scratch_types=[
        pltpu.VMEM((_CPS, _CH), jnp.int32),
        pltpu.VMEM((_CPS, _CH), jnp.int32),
        pltpu.VMEM((_CH, _D), jnp.float32),
        pltpu.VMEM((_CH, _D), jnp.float32),
        pltpu.VMEM((_CH, _D), jnp.float32),
        pltpu.VMEM((_CH, _D), jnp.float32),
        pltpu.VMEM_SHARED((_N, _D), jnp.float32),
        pltpu.SemaphoreType.DMA,
        pltpu.SemaphoreType.DMA,
        pltpu.SemaphoreType.DMA,
        pltpu.SemaphoreType.DMA,
        pltpu.SemaphoreType.DMA,
        pltpu.SemaphoreType.DMA,
        pltpu.SemaphoreType.DMA,
        pltpu.SemaphoreType.DMA,
    ],
)
def _msg_kernel(g_hbm, src_hbm, dst_hbm, msg_hbm,
                src_v, dst_v, rb0, rb1, rb2, rb3, out_acc,
                sg0, sg1, sg2, sg3, ss0, ss1, ss2, ss3):
    rows = [rb0, rb1, rb2, rb3]
    sg = [sg0, sg1, sg2, sg3]
    ss = [ss0, ss1, ss2, ss3]
    c = lax.axis_index("c")
    s = lax.axis_index("s")
    wid = c * _NS + s

    pltpu.make_async_copy(src_hbm.at[wid, 0], src_v, sg1).start()
    pltpu.make_async_copy(dst_hbm.at[wid, 0], dst_v, sg2).start()

    def zbody(i, carry):
        r = i // (_D // 16)
        k = i % (_D // 16)
        rb0[r, pl.ds(k * 16, 16)] = jnp.zeros((16,), jnp.float32)
        return carry

    lax.fori_loop(0, _CH * (_D // 16), zbody, 0)
    for i, (off, sz) in enumerate(_STRIPES):
        @pl.when(s == i)
        def _z(off=off, sz=sz):
            for k in range(sz // _CH):
                pltpu.sync_copy(rb0, out_acc.at[pl.ds(off + k * _CH, _CH)])
            tail = sz % _CH
            if tail:
                pltpu.sync_copy(
                    rb0.at[pl.ds(0, tail)],
                    out_acc.at[pl.ds(off + sz - tail, tail)])
    plsc.subcore_barrier()

    for sup in range(_NSUP):
        if sup == 0:
            pltpu.make_async_copy(src_hbm.at[wid, 0], src_v, sg1).wait()
            pltpu.make_async_copy(dst_hbm.at[wid, 0], dst_v, sg2).wait()
        else:
            pltpu.sync_copy(src_hbm.at[wid, sup], src_v)
            pltpu.sync_copy(dst_hbm.at[wid, sup], dst_v)
        for b in range(4):
            pltpu.make_async_copy(
                g_hbm.at[src_v.at[b]], rows[b], sg[b]).start()

        def body(i, carry):
            j0 = 4 * i
            for b in range(4):
                pltpu.make_async_copy(
                    g_hbm.at[src_v.at[j0 + b]], rows[b], sg[b]).wait()
                pltpu.make_async_copy(
                    rows[b], out_acc.at[dst_v.at[j0 + b]], ss[b]
                ).start(add=True)
            for b in range(4):
                pltpu.make_async_copy(
                    rows[b], out_acc.at[dst_v.at[j0 + b]], ss[b]).wait()
                pltpu.make_async_copy(
                    g_hbm.at[src_v.at[j0 + 4 + b]], rows[b], sg[b]).start()
            return carry

        lax.fori_loop(0, (_CPS - 5) // 4, body, 0)
        for b in range(4):
            j = _CPS - 5 + b
            pltpu.make_async_copy(
                g_hbm.at[src_v.at[j]], rows[b], sg[b]).wait()
            pltpu.make_async_copy(
                rows[b], out_acc.at[dst_v.at[j]], ss[b]).start(add=True)
        for b in range(4):
            pltpu.make_async_copy(
                rows[b], out_acc.at[dst_v.at[_CPS - 5 + b]], ss[b]).wait()
        pltpu.async_copy(
            g_hbm.at[src_v.at[_CPS - 1]], rows[0], sg[0]).wait()
        pltpu.sync_copy(rows[0], out_acc.at[dst_v.at[_CPS - 1]], add=True)
    plsc.subcore_barrier()
    for i, (off, sz) in enumerate(_STRIPES):
        @pl.when(s == i)
        def _cp(off=off, sz=sz):
            pltpu.sync_copy(out_acc.at[pl.ds(off, sz)],
                            msg_hbm.at[c, pl.ds(off, sz)])


def _elu(v):
    return jnp.where(v > 0, v, jnp.exp(jnp.minimum(v, 0.0)) - 1.0)


def _tc_first(deg_ref, x_ref, w_ref, dinv_ref, g_ref):
    deg = deg_ref[0] + deg_ref[1]
    dinv = jnp.where(deg > 0.0,
                     lax.rsqrt(jnp.maximum(deg, 1.0)),
                     0.0)
    dinv_ref[...] = dinv
    h = jnp.dot(x_ref[...], w_ref[...], preferred_element_type=jnp.float32)
    g_ref[...] = h * dinv


def _tc_mid(msg_ref, dinv_ref, b_ref, w_ref, g_ref):
    dinv = dinv_ref[...]
    m = msg_ref[0] + msg_ref[1]
    act = _elu(m * dinv + b_ref[...])
    h = jnp.dot(act, w_ref[...], preferred_element_type=jnp.float32)
    g_ref[...] = h * dinv


def _tc_last(msg_ref, dinv_ref, b_ref, out_ref):
    m = msg_ref[0] + msg_ref[1]
    out_ref[...] = _elu(m * dinv_ref[...] + b_ref[...])


_BR = 1000
_GRID = _N // _BR

_spec_msg = pl.BlockSpec((_NC, _BR, _D), lambda i: (0, i, 0))
_spec_deg = pl.BlockSpec((_NC, _BR, 1), lambda i: (0, i, 0))
_spec_rows = pl.BlockSpec((_BR, _D), lambda i: (i, 0))
_spec_dinv = pl.BlockSpec((_BR, 1), lambda i: (i, 0))
_spec_w = pl.BlockSpec((_D, _D), lambda i: (0, 0))
_spec_b = pl.BlockSpec((1, _D), lambda i: (0, 0))

_tc_first_call = pl.pallas_call(
    _tc_first,
    grid=(_GRID,),
    in_specs=[_spec_deg, _spec_rows, _spec_w],
    out_specs=(_spec_dinv, _spec_rows),
    out_shape=(
        jax.ShapeDtypeStruct((_N, 1), jnp.float32),
        jax.ShapeDtypeStruct((_N, _D), jnp.float32),
    ),
)

_tc_mid_call = pl.pallas_call(
    _tc_mid,
    grid=(_GRID,),
    in_specs=[_spec_msg, _spec_dinv, _spec_b, _spec_w],
    out_specs=_spec_rows,
    out_shape=jax.ShapeDtypeStruct((_N, _D), jnp.float32),
)

_tc_last_call = pl.pallas_call(
    _tc_last,
    grid=(_GRID,),
    in_specs=[_spec_msg, _spec_dinv, _spec_b],
    out_specs=_spec_rows,
    out_shape=jax.ShapeDtypeStruct((_N, _D), jnp.float32),
)


def kernel(x, edge_index, W1, b1, W2, b2, W3, b3):
    src_r = edge_index[0].reshape(_NW, _NSUP, _CPS, _CH)
    dst_r = edge_index[1].reshape(_NW, _NSUP, _CPS, _CH)

    deg_parts = _deg_kernel(dst_r)
    deg3 = deg_parts.reshape(_NC, _N, 1)

    dinv, g = _tc_first_call(deg3, x, W1)
    m = _msg_kernel(g, src_r, dst_r)
    g = _tc_mid_call(m, dinv, b1.reshape(1, _D), W2)
    m = _msg_kernel(g, src_r, dst_r)
    g = _tc_mid_call(m, dinv, b2.reshape(1, _D), W3)
    m = _msg_kernel(g, src_r, dst_r)
    return _tc_last_call(m, dinv, b3.reshape(1, _D))


# speedup vs baseline: 1.0206x; 1.0206x over previous
import functools

import jax
import jax.numpy as jnp
from jax import lax
from jax.experimental import pallas as pl
from jax.experimental.pallas import tpu as pltpu
from jax.experimental.pallas import tpu_sc as plsc

_N = 10000
_E = 320000
_D = 128

_NC = 2
_NS = 16
_NW = _NC * _NS
_EPW = _E // _NW
_CH = 80
_NCHUNK = _EPW // _CH
_NSUP = 5
_CPS = _NCHUNK // _NSUP

_STRIPE_BOUNDS = [8 * ((_N // 8) * i // _NS) for i in range(_NS)] + [_N]
_STRIPES = [(_STRIPE_BOUNDS[i], _STRIPE_BOUNDS[i + 1] - _STRIPE_BOUNDS[i])
            for i in range(_NS)]

_mesh = plsc.VectorSubcoreMesh(core_axis_name="c", subcore_axis_name="s")


@functools.partial(
    pl.kernel,
    out_type=jax.ShapeDtypeStruct((_NC, _N), jnp.float32),
    mesh=_mesh,
    scratch_types=[
        pltpu.VMEM((_NSUP, _CPS, _CH), jnp.int32),
        pltpu.VMEM((_CH,), jnp.float32),
        pltpu.VMEM((_N,), jnp.float32),
        pltpu.VMEM_SHARED((_N,), jnp.float32),
        pltpu.SemaphoreType.DMA,
    ],
)
def _deg_kernel(dst_hbm, deg_hbm, dst_v, ones_v, zeros_v, deg_acc, sem_d):
    c = lax.axis_index("c")
    s = lax.axis_index("s")
    wid = c * _NS + s
    pltpu.sync_copy(dst_hbm.at[wid], dst_v)
    for k in range(_CH // 16):
        ones_v[pl.ds(k * 16, 16)] = jnp.ones((16,), jnp.float32)

    @pl.when(s == 0)
    def _zero():
        def zbody(i, carry):
            zeros_v[pl.ds(i * 16, 16)] = jnp.zeros((16,), jnp.float32)
            return carry

        lax.fori_loop(0, _N // 16, zbody, 0)
        pltpu.sync_copy(zeros_v, deg_acc)

    plsc.subcore_barrier()

    for sup in range(_NSUP):
        def body(j, carry, sup=sup):
            pltpu.make_async_copy(
                ones_v, deg_acc.at[dst_v.at[sup, j]], sem_d).start(add=True)
            return carry

        lax.fori_loop(0, _CPS, body, 0)

    def drain(j, carry):
        pltpu.make_async_copy(
            ones_v, deg_acc.at[dst_v.at[0, 0]], sem_d).wait()
        return carry

    lax.fori_loop(0, _NCHUNK, drain, 0)
    plsc.subcore_barrier()

    @pl.when(s == 0)
    def _out():
        pltpu.sync_copy(deg_acc, deg_hbm.at[c])


@functools.partial(
    pl.kernel,
    out_type=jax.ShapeDtypeStruct((_NC, _N, _D), jnp.float32),
    mesh=_mesh,
    scratch_types=[
        pltpu.VMEM((_CPS, _CH), jnp.int32),
        pltpu.VMEM((_CPS, _CH), jnp.int32),
        pltpu.VMEM((_CH, _D), jnp.float32),
        pltpu.VMEM((_CH, _D), jnp.float32),
        pltpu.VMEM((_CH, _D), jnp.float32),
        pltpu.VMEM((_CH, _D), jnp.float32),
        pltpu.VMEM_SHARED((_N, _D), jnp.float32),
        pltpu.SemaphoreType.DMA,
        pltpu.SemaphoreType.DMA,
        pltpu.SemaphoreType.DMA,
        pltpu.SemaphoreType.DMA,
        pltpu.SemaphoreType.DMA,
        pltpu.SemaphoreType.DMA,
        pltpu.SemaphoreType.DMA,
        pltpu.SemaphoreType.DMA,
    ],
)
def _msg_kernel(g_hbm, src_hbm, dst_hbm, msg_hbm,
                src_v, dst_v, rb0, rb1, rb2, rb3, out_acc,
                sg0, sg1, sg2, sg3, ss0, ss1, ss2, ss3):
    rows = [rb0, rb1, rb2, rb3]
    sg = [sg0, sg1, sg2, sg3]
    ss = [ss0, ss1, ss2, ss3]
    c = lax.axis_index("c")
    s = lax.axis_index("s")
    wid = c * _NS + s

    pltpu.make_async_copy(src_hbm.at[wid, 0], src_v, sg1).start()
    pltpu.make_async_copy(dst_hbm.at[wid, 0], dst_v, sg2).start()

    def zbody(i, carry):
        r = i // (_D // 16)
        k = i % (_D // 16)
        rb0[r, pl.ds(k * 16, 16)] = jnp.zeros((16,), jnp.float32)
        return carry

    lax.fori_loop(0, _CH * (_D // 16), zbody, 0)
    for i, (off, sz) in enumerate(_STRIPES):
        @pl.when(s == i)
        def _z(off=off, sz=sz):
            for k in range(sz // _CH):
                pltpu.sync_copy(rb0, out_acc.at[pl.ds(off + k * _CH, _CH)])
            tail = sz % _CH
            if tail:
                pltpu.sync_copy(
                    rb0.at[pl.ds(0, tail)],
                    out_acc.at[pl.ds(off + sz - tail, tail)])
    plsc.subcore_barrier()

    for sup in range(_NSUP):
        if sup == 0:
            pltpu.make_async_copy(src_hbm.at[wid, 0], src_v, sg1).wait()
            pltpu.make_async_copy(dst_hbm.at[wid, 0], dst_v, sg2).wait()
        else:
            pltpu.sync_copy(src_hbm.at[wid, sup], src_v)
            pltpu.sync_copy(dst_hbm.at[wid, sup], dst_v)
        for b in range(4):
            pltpu.make_async_copy(
                g_hbm.at[src_v.at[b]], rows[b], sg[b]).start()

        def body(i, carry):
            j0 = 4 * i
            for b in range(4):
                pltpu.make_async_copy(
                    g_hbm.at[src_v.at[j0 + b]], rows[b], sg[b]).wait()
                pltpu.make_async_copy(
                    rows[b], out_acc.at[dst_v.at[j0 + b]], ss[b]
                ).start(add=True)
            for b in range(4):
                pltpu.make_async_copy(
                    rows[b], out_acc.at[dst_v.at[j0 + b]], ss[b]).wait()
                pltpu.make_async_copy(
                    g_hbm.at[src_v.at[j0 + 4 + b]], rows[b], sg[b]).start()
            return carry

        lax.fori_loop(0, (_CPS - 5) // 4, body, 0)
        for b in range(4):
            j = _CPS - 5 + b
            pltpu.make_async_copy(
                g_hbm.at[src_v.at[j]], rows[b], sg[b]).wait()
            pltpu.make_async_copy(
                rows[b], out_acc.at[dst_v.at[j]], ss[b]).start(add=True)
        for b in range(4):
            pltpu.make_async_copy(
                rows[b], out_acc.at[dst_v.at[_CPS - 5 + b]], ss[b]).wait()
        pltpu.async_copy(
            g_hbm.at[src_v.at[_CPS - 1]], rows[0], sg[0]).wait()
        pltpu.sync_copy(rows[0], out_acc.at[dst_v.at[_CPS - 1]], add=True)
    plsc.subcore_barrier()
    for i, (off, sz) in enumerate(_STRIPES):
        @pl.when(s == i)
        def _cp(off=off, sz=sz):
            pltpu.sync_copy(out_acc.at[pl.ds(off, sz)],
                            msg_hbm.at[c, pl.ds(off, sz)])


def _elu(v):
    return jnp.where(v > 0, v, jnp.exp(jnp.minimum(v, 0.0)) - 1.0)


def _tc_matmul(x_ref, w_ref, h_ref):
    h_ref[...] = jnp.dot(x_ref[...], w_ref[...],
                         preferred_element_type=jnp.float32)


def _tc_scale(deg_ref, h_ref, dinv_ref, g_ref):
    deg = deg_ref[0] + deg_ref[1]
    dinv = jnp.where(deg > 0.0,
                     lax.rsqrt(jnp.maximum(deg, 1.0)),
                     0.0)
    dinv_ref[...] = dinv
    g_ref[...] = h_ref[...] * dinv


def _tc_mid(msg_ref, dinv_ref, b_ref, w_ref, g_ref):
    dinv = dinv_ref[...]
    m = msg_ref[0] + msg_ref[1]
    act = _elu(m * dinv + b_ref[...])
    h = jnp.dot(act, w_ref[...], preferred_element_type=jnp.float32)
    g_ref[...] = h * dinv


def _tc_last(msg_ref, dinv_ref, b_ref, out_ref):
    m = msg_ref[0] + msg_ref[1]
    out_ref[...] = _elu(m * dinv_ref[...] + b_ref[...])


_tc_matmul_call = pl.pallas_call(
    _tc_matmul,
    out_shape=jax.ShapeDtypeStruct((_N, _D), jnp.float32),
)

_tc_scale_call = pl.pallas_call(
    _tc_scale,
    out_shape=(
        jax.ShapeDtypeStruct((_N, 1), jnp.float32),
        jax.ShapeDtypeStruct((_N, _D), jnp.float32),
    ),
)

_tc_mid_call = pl.pallas_call(
    _tc_mid,
    out_shape=jax.ShapeDtypeStruct((_N, _D), jnp.float32),
)

_tc_last_call = pl.pallas_call(
    _tc_last,
    out_shape=jax.ShapeDtypeStruct((_N, _D), jnp.float32),
)


def kernel(x, edge_index, W1, b1, W2, b2, W3, b3):
    src_r = edge_index[0].reshape(_NW, _NSUP, _CPS, _CH)
    dst_r = edge_index[1].reshape(_NW, _NSUP, _CPS, _CH)

    deg_parts = _deg_kernel(dst_r)
    h1 = _tc_matmul_call(x, W1)
    deg3 = deg_parts.reshape(_NC, _N, 1)

    dinv, g = _tc_scale_call(deg3, h1)
    m = _msg_kernel(g, src_r, dst_r)
    g = _tc_mid_call(m, dinv, b1.reshape(1, _D), W2)
    m = _msg_kernel(g, src_r, dst_r)
    g = _tc_mid_call(m, dinv, b2.reshape(1, _D), W3)
    m = _msg_kernel(g, src_r, dst_r)
    return _tc_last_call(m, dinv, b3.reshape(1, _D))
